# Initial kernel scaffold; baseline (speedup 1.0000x reference)
import jax
import jax.numpy as jnp
from jax.experimental import pallas as pl


def kernel(x, wq_t, bq, wk_t, bk, wv_t, bv, wp_t, bp, wsr_t, bsr, ln_g, ln_b):
    raise NotImplementedError("write your pallas kernel here")


# speedup vs baseline: 2.6984x; 2.6984x over previous
import jax
import jax.numpy as jnp
from jax.experimental import pallas as pl
from jax.experimental.pallas import tpu as pltpu


def _fused_kernel(xn_ref, xp_ref, wsr_ref, bsr_ref, g_ref, beta_ref,
                  wq_ref, bq_ref, wkv_ref, bkv_ref, wp_ref, bp_ref, o_ref,
                  *, Hs, Ws, sr, C, N, tq, scale, eps):
    f32 = jnp.float32

    cols = []
    for dh in range(sr):
        rows = [xp_ref[0, hs, dh] for hs in range(Hs)]
        cols.append(jnp.concatenate(rows, axis=0))
    pmat = jnp.concatenate(cols, axis=1)

    y = jnp.dot(pmat, wsr_ref[...], preferred_element_type=f32) + bsr_ref[...]
    mu = jnp.mean(y, axis=-1, keepdims=True)
    yc = y - mu
    var = jnp.mean(yc * yc, axis=-1, keepdims=True)
    xs = yc * jax.lax.rsqrt(var + eps) * g_ref[...] + beta_ref[...]

    kv = jnp.dot(xs, wkv_ref[...], preferred_element_type=f32) + bkv_ref[...]
    k = kv[:, :C]
    v = kv[:, C:]

    n_qt = N // tq
    for qt in range(n_qt):
        xt = xn_ref[0, pl.ds(qt * tq, tq), :]
        q = (jnp.dot(xt, wq_ref[...], preferred_element_type=f32)
             + bq_ref[...]) * scale
        s = jax.lax.dot_general(q, k, (((1,), (1,)), ((), ())),
                                preferred_element_type=f32)
        m = jnp.max(s, axis=-1, keepdims=True)
        p = jnp.exp(s - m)
        l = jnp.sum(p, axis=-1, keepdims=True)
        o = jnp.dot(p, v, preferred_element_type=f32) * pl.reciprocal(l, approx=True)
        res = jnp.dot(o, wp_ref[...], preferred_element_type=f32) + bp_ref[...]
        o_ref[0, pl.ds(qt * tq, tq), :] = res.astype(o_ref.dtype)


def kernel(x, wq_t, bq, wk_t, bk, wv_t, bv, wp_t, bp, wsr_t, bsr, ln_g, ln_b):
    B, N, C = x.shape
    H = W = 56
    sr = 8
    Hs, Ws = H // sr, W // sr
    Nk = Hs * Ws
    scale = float(C) ** -0.5
    tq = 448
    import functools

    xp = x.reshape(B, Hs, sr, Ws, sr * C)
    wsr_r = wsr_t.reshape(C, sr, sr, C).transpose(1, 2, 0, 3).reshape(sr * sr * C, C)
    wkv = jnp.concatenate([wk_t, wv_t], axis=1)
    bkv = jnp.concatenate([bk, bv]).reshape(1, 2 * C)

    body = functools.partial(_fused_kernel, Hs=Hs, Ws=Ws, sr=sr, C=C, N=N,
                             tq=tq, scale=scale, eps=1e-5)

    return pl.pallas_call(
        body,
        out_shape=jax.ShapeDtypeStruct((B, N, C), x.dtype),
        grid=(B,),
        in_specs=[
            pl.BlockSpec((1, N, C), lambda b: (b, 0, 0)),
            pl.BlockSpec((1, Hs, sr, Ws, sr * C), lambda b: (b, 0, 0, 0, 0)),
            pl.BlockSpec((sr * sr * C, C), lambda b: (0, 0)),
            pl.BlockSpec((1, C), lambda b: (0, 0)),
            pl.BlockSpec((1, C), lambda b: (0, 0)),
            pl.BlockSpec((1, C), lambda b: (0, 0)),
            pl.BlockSpec((C, C), lambda b: (0, 0)),
            pl.BlockSpec((1, C), lambda b: (0, 0)),
            pl.BlockSpec((C, 2 * C), lambda b: (0, 0)),
            pl.BlockSpec((1, 2 * C), lambda b: (0, 0)),
            pl.BlockSpec((C, C), lambda b: (0, 0)),
            pl.BlockSpec((1, C), lambda b: (0, 0)),
        ],
        out_specs=pl.BlockSpec((1, N, C), lambda b: (b, 0, 0)),
        compiler_params=pltpu.CompilerParams(
            dimension_semantics=("parallel",),
            vmem_limit_bytes=64 * 1024 * 1024,
        ),
    )(x, xp, wsr_r, bsr.reshape(1, C), ln_g.reshape(1, C), ln_b.reshape(1, C),
      wq_t, bq.reshape(1, C), wkv, bkv, wp_t, bp.reshape(1, C))


# speedup vs baseline: 3.3138x; 1.2281x over previous
import functools

import jax
import jax.numpy as jnp
from jax.experimental import pallas as pl
from jax.experimental.pallas import tpu as pltpu


def _fused_kernel(xn_ref, xp_ref, wsr_ref, bsr_ref, wq_ref, bq_ref,
                  wkv_ref, bkv_ref, wp_ref, bp_ref, o_ref,
                  *, Hs, sr, C, N, tq, eps, bb):
    f32 = jnp.float32

    for i in range(bb):
        cols = []
        for dh in range(sr):
            rows = [xp_ref[i, hs, dh] for hs in range(Hs)]
            cols.append(jnp.concatenate(rows, axis=0))
        pmat = jnp.concatenate(cols, axis=1)

        y = jnp.dot(pmat, wsr_ref[...], preferred_element_type=f32) + bsr_ref[...]
        mu = jnp.mean(y, axis=-1, keepdims=True)
        yc = y - mu
        var = jnp.mean(yc * yc, axis=-1, keepdims=True)
        xs = yc * jax.lax.rsqrt(var + eps)

        kv = jnp.dot(xs, wkv_ref[...], preferred_element_type=f32) + bkv_ref[...]
        k = kv[:, :C]
        v = kv[:, C:]
        kbq = jnp.dot(k, bq_ref[...], preferred_element_type=f32)

        n_qt = N // tq
        for qt in range(n_qt):
            xt = xn_ref[i, pl.ds(qt * tq, tq), :]
            qT = jax.lax.dot_general(wq_ref[...], xt, (((0,), (1,)), ((), ())),
                                     preferred_element_type=f32)
            s = jax.lax.dot_general(k, qT, (((1,), (0,)), ((), ())),
                                    preferred_element_type=f32) + kbq
            m = jnp.max(s, axis=0, keepdims=True)
            p = jnp.exp(s - m)
            l = jnp.sum(p, axis=0, keepdims=True)
            oT = jax.lax.dot_general(v, p, (((0,), (0,)), ((), ())),
                                     preferred_element_type=f32)
            oT = oT * pl.reciprocal(l, approx=True)
            res = jax.lax.dot_general(oT, wp_ref[...], (((0,), (0,)), ((), ())),
                                      preferred_element_type=f32) + bp_ref[...]
            o_ref[i, pl.ds(qt * tq, tq), :] = res.astype(o_ref.dtype)


def kernel(x, wq_t, bq, wk_t, bk, wv_t, bv, wp_t, bp, wsr_t, bsr, ln_g, ln_b):
    B, N, C = x.shape
    H = W = 56
    sr = 8
    Hs, Ws = H // sr, W // sr
    scale = float(C) ** -0.5
    tq = 3136

    xp = x.reshape(B, Hs, sr, Ws, sr * C)
    wsr_r = wsr_t.reshape(C, sr, sr, C).transpose(1, 2, 0, 3).reshape(sr * sr * C, C)
    wq_s = wq_t * scale
    bq_s = (bq * scale).reshape(C, 1)
    wkv = jnp.concatenate([wk_t, wv_t], axis=1) * ln_g.reshape(C, 1)
    bkv = (jnp.concatenate([bk, bv])
           + jnp.dot(ln_b, jnp.concatenate([wk_t, wv_t], axis=1),
                     precision=jax.lax.Precision.HIGHEST)).reshape(1, 2 * C)

    bb = 1
    body = functools.partial(_fused_kernel, Hs=Hs, sr=sr, C=C, N=N,
                             tq=tq, eps=1e-5, bb=bb)

    return pl.pallas_call(
        body,
        out_shape=jax.ShapeDtypeStruct((B, N, C), x.dtype),
        grid=(B // bb,),
        in_specs=[
            pl.BlockSpec((bb, N, C), lambda b: (b, 0, 0)),
            pl.BlockSpec((bb, Hs, sr, Ws, sr * C), lambda b: (b, 0, 0, 0, 0)),
            pl.BlockSpec((sr * sr * C, C), lambda b: (0, 0)),
            pl.BlockSpec((1, C), lambda b: (0, 0)),
            pl.BlockSpec((C, C), lambda b: (0, 0)),
            pl.BlockSpec((C, 1), lambda b: (0, 0)),
            pl.BlockSpec((C, 2 * C), lambda b: (0, 0)),
            pl.BlockSpec((1, 2 * C), lambda b: (0, 0)),
            pl.BlockSpec((C, C), lambda b: (0, 0)),
            pl.BlockSpec((1, C), lambda b: (0, 0)),
        ],
        out_specs=pl.BlockSpec((bb, N, C), lambda b: (b, 0, 0)),
        compiler_params=pltpu.CompilerParams(
            dimension_semantics=("parallel",),
            vmem_limit_bytes=64 * 1024 * 1024,
        ),
    )(x, xp, wsr_r, bsr.reshape(1, C), wq_s, bq_s, wkv, bkv, wp_t,
      bp.reshape(1, C))


# speedup vs baseline: 3.5984x; 1.0859x over previous
import functools

import jax
import jax.numpy as jnp
from jax.experimental import pallas as pl
from jax.experimental.pallas import tpu as pltpu


def _fused_kernel(xn_ref, xp_ref, wsr_ref, bsr_ref, wq_ref, bq_ref,
                  wkv_ref, bkv_ref, wp_ref, bp_ref, o_ref,
                  *, Hs, sr, C, N, tq, eps, bb):
    f32 = jnp.float32

    for i in range(bb):
        cols = []
        for dh in range(sr):
            rows = [xp_ref[i, hs, dh] for hs in range(Hs)]
            cols.append(jnp.concatenate(rows, axis=0))
        pmat = jnp.concatenate(cols, axis=1)

        y = jnp.dot(pmat, wsr_ref[...], preferred_element_type=f32) + bsr_ref[...]
        mu = jnp.mean(y, axis=-1, keepdims=True)
        yc = y - mu
        var = jnp.mean(yc * yc, axis=-1, keepdims=True)
        xs = yc * jax.lax.rsqrt(var + eps)

        kv = jnp.dot(xs, wkv_ref[...], preferred_element_type=f32) + bkv_ref[...]
        k = kv[:, :C]
        v = kv[:, C:]
        kbq = jnp.dot(k, bq_ref[...], preferred_element_type=f32)

        n_qt = N // tq
        for qt in range(n_qt):
            xt = xn_ref[i, pl.ds(qt * tq, tq), :]
            qT = jax.lax.dot_general(wq_ref[...], xt, (((0,), (1,)), ((), ())),
                                     preferred_element_type=f32)
            s = jax.lax.dot_general(k, qT, (((1,), (0,)), ((), ())),
                                    preferred_element_type=f32) + kbq
            m = jnp.max(s, axis=0, keepdims=True)
            p = jnp.exp(s - m)
            l = jnp.sum(p, axis=0, keepdims=True)
            oT = jax.lax.dot_general(v, p, (((0,), (0,)), ((), ())),
                                     preferred_element_type=f32)
            oT = oT * pl.reciprocal(l, approx=True)
            res = jax.lax.dot_general(oT, wp_ref[...], (((0,), (0,)), ((), ())),
                                      preferred_element_type=f32) + bp_ref[...]
            o_ref[i, pl.ds(qt * tq, tq), :] = res.astype(o_ref.dtype)


def kernel(x, wq_t, bq, wk_t, bk, wv_t, bv, wp_t, bp, wsr_t, bsr, ln_g, ln_b):
    B, N, C = x.shape
    H = W = 56
    sr = 8
    Hs, Ws = H // sr, W // sr
    scale = float(C) ** -0.5
    tq = 3136

    xp = x.reshape(B, Hs, sr, Ws, sr * C)
    wsr_r = wsr_t.reshape(C, sr, sr, C).transpose(1, 2, 0, 3).reshape(sr * sr * C, C)
    wq_s = wq_t * scale
    bq_s = (bq * scale).reshape(C, 1)
    wkv = jnp.concatenate([wk_t, wv_t], axis=1) * ln_g.reshape(C, 1)
    bkv = (jnp.concatenate([bk, bv])
           + jnp.dot(ln_b, jnp.concatenate([wk_t, wv_t], axis=1),
                     precision=jax.lax.Precision.HIGHEST)).reshape(1, 2 * C)

    bb = 4
    body = functools.partial(_fused_kernel, Hs=Hs, sr=sr, C=C, N=N,
                             tq=tq, eps=1e-5, bb=bb)

    return pl.pallas_call(
        body,
        out_shape=jax.ShapeDtypeStruct((B, N, C), x.dtype),
        grid=(B // bb,),
        in_specs=[
            pl.BlockSpec((bb, N, C), lambda b: (b, 0, 0)),
            pl.BlockSpec((bb, Hs, sr, Ws, sr * C), lambda b: (b, 0, 0, 0, 0)),
            pl.BlockSpec((sr * sr * C, C), lambda b: (0, 0)),
            pl.BlockSpec((1, C), lambda b: (0, 0)),
            pl.BlockSpec((C, C), lambda b: (0, 0)),
            pl.BlockSpec((C, 1), lambda b: (0, 0)),
            pl.BlockSpec((C, 2 * C), lambda b: (0, 0)),
            pl.BlockSpec((1, 2 * C), lambda b: (0, 0)),
            pl.BlockSpec((C, C), lambda b: (0, 0)),
            pl.BlockSpec((1, C), lambda b: (0, 0)),
        ],
        out_specs=pl.BlockSpec((bb, N, C), lambda b: (b, 0, 0)),
        compiler_params=pltpu.CompilerParams(
            dimension_semantics=("parallel",),
            vmem_limit_bytes=64 * 1024 * 1024,
        ),
    )(x, xp, wsr_r, bsr.reshape(1, C), wq_s, bq_s, wkv, bkv, wp_t,
      bp.reshape(1, C))
